# Initial kernel scaffold; baseline (speedup 1.0000x reference)
import jax
import jax.numpy as jnp
from jax.experimental import pallas as pl


def kernel(images):
    raise NotImplementedError("write your pallas kernel here")


# speedup vs baseline: 9.5933x; 9.5933x over previous
import functools

import jax
import jax.numpy as jnp
from jax.experimental import pallas as pl

_B = 16384
_IJ = 32


def _dp_body(x_ref, o_ref):
    x = x_ref[...]
    t = jax.nn.softplus(x)
    tt = t.T
    inf_row = jnp.full((1, tt.shape[1]), jnp.inf, jnp.float32)
    d = tt[0:_IJ, :]
    for i in range(1, _IJ):
        th = tt[_IJ * i:_IJ * (i + 1), :]
        up = jnp.concatenate([inf_row, d[:-1, :]], axis=0)
        dn = jnp.concatenate([d[1:, :], inf_row], axis=0)
        d = th + jnp.minimum(d, jnp.minimum(up, dn))
    o_ref[...] = jnp.min(d, axis=0)


def _run(images, interpret):
    x = images.reshape(_B, _IJ * _IJ)
    tb = 512
    return pl.pallas_call(
        _dp_body,
        grid=(_B // tb,),
        in_specs=[pl.BlockSpec((tb, _IJ * _IJ), lambda b: (b, 0))],
        out_specs=pl.BlockSpec((tb,), lambda b: (b,)),
        out_shape=jax.ShapeDtypeStruct((_B,), jnp.float32),
        interpret=interpret,
    )(x)


@jax.jit
def kernel(images):
    return _run(images, False)


# speedup vs baseline: 10.3908x; 1.0831x over previous
import functools

import jax
import jax.numpy as jnp
from jax.experimental import pallas as pl
from jax.experimental.pallas import tpu as pltpu

_B = 16384
_IJ = 32


_LOG2E = 1.4426950408889634
_LN2 = 0.6931471805599453


def _softplus(x):
    a = jnp.abs(x)
    u = jnp.exp2(a * (-_LOG2E))
    return jnp.maximum(x, 0.0) + jnp.log2(1.0 + u) * _LN2


def _dp_body(x_ref, o_ref):
    x = x_ref[...]
    t = _softplus(x)
    tt = t.T
    inf_row = jnp.full((1, tt.shape[1]), jnp.inf, jnp.float32)
    d = tt[0:_IJ, :]
    for i in range(1, _IJ):
        th = tt[_IJ * i:_IJ * (i + 1), :]
        up = jnp.concatenate([inf_row, d[:-1, :]], axis=0)
        dn = jnp.concatenate([d[1:, :], inf_row], axis=0)
        d = th + jnp.minimum(d, jnp.minimum(up, dn))
    o_ref[...] = jnp.min(d, axis=0)


def _run(images, interpret):
    x = images.reshape(_B, _IJ * _IJ)
    tb = 512
    return pl.pallas_call(
        _dp_body,
        grid=(_B // tb,),
        in_specs=[pl.BlockSpec((tb, _IJ * _IJ), lambda b: (b, 0))],
        out_specs=pl.BlockSpec((tb,), lambda b: (b,)),
        out_shape=jax.ShapeDtypeStruct((_B,), jnp.float32),
        interpret=interpret,
    )(x)


@jax.jit
def kernel(images):
    return _run(images, False)


# speedup vs baseline: 22.7685x; 2.1912x over previous
import functools

import jax
import jax.numpy as jnp
from jax.experimental import pallas as pl
from jax.experimental.pallas import tpu as pltpu

_B = 16384
_IJ = 32


_LOG2E = 1.4426950408889634
_LN2 = 0.6931471805599453


def _softplus(x):
    a = jnp.abs(x)
    u = jnp.exp2(a * (-_LOG2E))
    return jnp.maximum(x, 0.0) + jnp.log2(1.0 + u) * _LN2


def _dp_body(x_ref, o_ref):
    tb = x_ref.shape[2]
    inf_row = jnp.full((1, tb), jnp.inf, jnp.float32)
    d = _softplus(x_ref[0])
    for i in range(1, _IJ):
        up = jnp.concatenate([inf_row, d[:-1, :]], axis=0)
        dn = jnp.concatenate([d[1:, :], inf_row], axis=0)
        d = _softplus(x_ref[i]) + jnp.minimum(d, jnp.minimum(up, dn))
    o_ref[...] = jnp.min(d, axis=0)


def _run(images, interpret):
    tb = 512
    y = jnp.moveaxis(images, 0, 2)
    return pl.pallas_call(
        _dp_body,
        grid=(_B // tb,),
        in_specs=[pl.BlockSpec((_IJ, _IJ, tb), lambda b: (0, 0, b))],
        out_specs=pl.BlockSpec((tb,), lambda b: (b,)),
        out_shape=jax.ShapeDtypeStruct((_B,), jnp.float32),
        interpret=interpret,
    )(y)


@jax.jit
def kernel(images):
    return _run(images, False)
